# baseline (speedup 1.0000x reference)
import functools

import jax
import jax.numpy as jnp
from jax import lax
from jax.experimental import pallas as pl
from jax.experimental.pallas import tpu as pltpu
from jax.experimental.pallas import tpu_sc as plsc

NC, NS, L = 2, 16, 16
NW = NC * NS

B = 4
HF = 512
OC = 16
H0 = 128
NUM_CLASS = 21
NUM_OLD = 16

_mesh = lambda: plsc.VectorSubcoreMesh(core_axis_name="c", subcore_axis_name="s")
_SC_PARAMS = pltpu.CompilerParams(use_tc_tiling_on_sc=False,
                                  needs_layout_passes=False)


def _dsum_body(f_ref, g_ref, o_ref):
    c = pl.program_id(1)

    @pl.when(c == 0)
    def _():
        o_ref[...] = jnp.zeros_like(o_ref)

    x = f_ref[...] - g_ref[...]
    o_ref[...] += jnp.sum(x * x, axis=1, keepdims=True)


def _dsum(f, f_old, c_blk):
    b, c, h, w = f.shape
    out = pl.pallas_call(
        _dsum_body,
        grid=(b, c // c_blk),
        in_specs=[
            pl.BlockSpec((1, c_blk, h, w), lambda i, j: (i, j, 0, 0)),
            pl.BlockSpec((1, c_blk, h, w), lambda i, j: (i, j, 0, 0)),
        ],
        out_specs=pl.BlockSpec((1, 1, h, w), lambda i, j: (i, 0, 0, 0)),
        out_shape=jax.ShapeDtypeStruct((b, 1, h, w), jnp.float32),
        compiler_params=pltpu.CompilerParams(
            dimension_semantics=("parallel", "arbitrary")),
    )(f, f_old)
    return out.reshape(b, h * w)


def _dsum_cl_body(f_ref, g_ref, o_ref):
    x = f_ref[...] - g_ref[...]
    o_ref[0] = jnp.sum(x * x, axis=2)


def _dsum_cl(f, f_old, hw_blk):
    b, c, h, w = f.shape
    hw = h * w
    f2 = f.transpose(0, 2, 3, 1).reshape(b, hw, c)
    g2 = f_old.transpose(0, 2, 3, 1).reshape(b, hw, c)
    out = pl.pallas_call(
        _dsum_cl_body,
        grid=(b, hw // hw_blk),
        in_specs=[
            pl.BlockSpec((1, hw_blk, c), lambda i, j: (i, j, 0)),
            pl.BlockSpec((1, hw_blk, c), lambda i, j: (i, j, 0)),
        ],
        out_specs=pl.BlockSpec((1, 1, hw_blk), lambda i, j: (i, 0, j)),
        out_shape=jax.ShapeDtypeStruct((b, 1, hw), jnp.float32),
        compiler_params=pltpu.CompilerParams(
            dimension_semantics=("parallel", "parallel")),
    )(f2, g2)
    return out.reshape(b, hw)


_RB = 64


def _pseudo_body(oo_ref, lab_ref, o_ref):
    v = jnp.concatenate(
        [oo_ref[0, :, 4 * k:4 * k + 1, :] for k in range(_RB // 4)], axis=1)
    v = jnp.where(v < 0.5, jnp.float32(0.0), v)
    best = v[0]
    bidx = jnp.zeros((_RB // 4, HF), jnp.int32)
    for c in range(1, OC):
        take = v[c] > best
        best = jnp.where(take, v[c], best)
        bidx = jnp.where(take, c, bidx)
    lab = jnp.concatenate(
        [lab_ref[0, 4 * k:4 * k + 1, :] for k in range(_RB // 4)], axis=0)
    o_ref[0, 0] = jnp.where(lab == 0, bidx, lab)


def _pseudo_tc(outputs_old, labels):
    out = pl.pallas_call(
        _pseudo_body,
        grid=(B, HF // _RB),
        in_specs=[
            pl.BlockSpec((1, OC, _RB, HF), lambda b, r: (b, 0, r, 0)),
            pl.BlockSpec((1, _RB, HF), lambda b, r: (b, r, 0)),
        ],
        out_specs=pl.BlockSpec((1, 1, _RB // 4, HF), lambda b, r: (b, r, 0, 0)),
        out_shape=jax.ShapeDtypeStruct((B, HF // _RB, _RB // 4, HF),
                                       jnp.int32),
        compiler_params=pltpu.CompilerParams(
            dimension_semantics=("parallel", "parallel")),
    )(outputs_old, labels)
    return out.reshape(B, H0 * HF)


def _px_group(fbuf, gbuf, nsub, cch, iota):
    bases = [((s * L + iota) * cch) for s in range(nsub)]
    zeros = tuple(jnp.zeros((L,), jnp.float32) for _ in range(nsub))

    def jbody(j, accs):
        out = []
        for s in range(nsub):
            a = plsc.load_gather(fbuf, [bases[s] + j])
            b_ = plsc.load_gather(gbuf, [bases[s] + j])
            x = a - b_
            out.append(accs[s] + x * x)
        return tuple(out)

    return lax.fori_loop(0, cch, jbody, zeros)


@functools.partial(
    pl.kernel,
    out_type=(
        jax.ShapeDtypeStruct((16384,), jnp.float32),
        jax.ShapeDtypeStruct((4096,), jnp.float32),
        jax.ShapeDtypeStruct((1024,), jnp.float32),
    ),
    mesh=_mesh(),
    scratch_types=[
        pltpu.VMEM((128 * 256,), jnp.float32),
        pltpu.VMEM((128 * 256,), jnp.float32),
        pltpu.VMEM((512,), jnp.float32),
        pltpu.VMEM((128,), jnp.float32),
        pltpu.VMEM((32,), jnp.float32),
    ],
    compiler_params=_SC_PARAMS,
)
def _dsum_sc(f1_hbm, g1_hbm, f2_hbm, g2_hbm, f3_hbm, g3_hbm,
             d1_hbm, d2_hbm, d3_hbm, fbuf, gbuf, o1_v, o2_v, o3_v):
    wid = lax.axis_index("s") * NC + lax.axis_index("c")
    iota = lax.iota(jnp.int32, L)

    for g in range(4):
        base = (wid * 512 + g * 128) * 256
        pltpu.sync_copy(f1_hbm.at[pl.ds(base, 128 * 256)], fbuf)
        pltpu.sync_copy(g1_hbm.at[pl.ds(base, 128 * 256)], gbuf)
        accs = _px_group(fbuf, gbuf, 8, 256, iota)
        for s in range(8):
            o1_v[pl.ds(g * 128 + s * L, L)] = accs[s]
    pltpu.sync_copy(o1_v, d1_hbm.at[pl.ds(wid * 512, 512)])

    for g in range(2):
        base = (wid * 128 + g * 64) * 512
        pltpu.sync_copy(f2_hbm.at[pl.ds(base, 64 * 512)],
                        fbuf.at[pl.ds(0, 64 * 512)])
        pltpu.sync_copy(g2_hbm.at[pl.ds(base, 64 * 512)],
                        gbuf.at[pl.ds(0, 64 * 512)])
        accs = _px_group(fbuf, gbuf, 4, 512, iota)
        for s in range(4):
            o2_v[pl.ds(g * 64 + s * L, L)] = accs[s]
    pltpu.sync_copy(o2_v, d2_hbm.at[pl.ds(wid * 128, 128)])

    base = wid * 32 * 512
    pltpu.sync_copy(f3_hbm.at[pl.ds(base, 32 * 512)],
                    fbuf.at[pl.ds(0, 32 * 512)])
    pltpu.sync_copy(g3_hbm.at[pl.ds(base, 32 * 512)],
                    gbuf.at[pl.ds(0, 32 * 512)])
    accs = _px_group(fbuf, gbuf, 2, 512, iota)
    for s in range(2):
        o3_v[pl.ds(s * L, L)] = accs[s]
    pltpu.sync_copy(o3_v, d3_hbm.at[pl.ds(wid * 32, 32)])


@functools.partial(
    pl.kernel,
    out_type=(
        jax.ShapeDtypeStruct((NW * 32,), jnp.float32),
        jax.ShapeDtypeStruct((NW * 32,), jnp.float32),
    ),
    mesh=_mesh(),
    scratch_types=[
        pltpu.VMEM((16 * HF,), jnp.int32),
        pltpu.VMEM((2048,), jnp.float32),
        pltpu.VMEM((32,), jnp.float32),
        pltpu.VMEM((32,), jnp.float32),
    ],
    compiler_params=_SC_PARAMS,
)
def _bin0_kernel(ps_hbm, d0_hbm, s_hbm, n_hbm, ps_v, d_v, s_v, n_v):
    wid = lax.axis_index("s") * NC + lax.axis_index("c")
    b = wid // 8
    seg = wid % 8
    zero = jnp.zeros((L,), jnp.float32)
    ones = jnp.ones((L,), jnp.float32)
    iota = lax.iota(jnp.int32, L)
    s_v[pl.ds(0, L)] = zero
    s_v[pl.ds(L, L)] = zero
    n_v[pl.ds(0, L)] = zero
    n_v[pl.ds(L, L)] = zero
    pltpu.sync_copy(ps_hbm.at[b, pl.ds(seg * (16 * HF), 16 * HF)], ps_v)
    pltpu.sync_copy(d0_hbm.at[b, pl.ds(seg * 2048, 2048)], d_v)

    def s0_body(v, carry):
        q = v * L + iota
        pidx = ((q >> 7) << 9) + ((q & 127) << 2)
        p = plsc.load_gather(ps_v, [pidx])
        d = d_v[pl.ds(v * L, L)]
        plsc.addupdate_scatter(s_v, [p], d)
        plsc.addupdate_scatter(n_v, [p], ones)
        return carry

    lax.fori_loop(0, 2048 // L, s0_body, 0)
    pltpu.sync_copy(s_v, s_hbm.at[pl.ds(wid * 32, 32)])
    pltpu.sync_copy(n_v, n_hbm.at[pl.ds(wid * 32, 32)])


@functools.partial(
    pl.kernel,
    out_type=jax.ShapeDtypeStruct((L,), jnp.float32),
    mesh=_mesh(),
    scratch_types=[
        pltpu.VMEM((32 * HF,), jnp.int32),
        pltpu.VMEM((1024,), jnp.float32),
        pltpu.VMEM((128,), jnp.float32),
        pltpu.VMEM((128,), jnp.float32),
        pltpu.VMEM_SHARED((NS, 128), jnp.float32),
        pltpu.VMEM_SHARED((NS, 128), jnp.float32),
        pltpu.VMEM((NS, 128), jnp.float32),
        pltpu.VMEM((NS, 128), jnp.float32),
        pltpu.VMEM((NW * 32,), jnp.float32),
        pltpu.VMEM((NW * 32,), jnp.float32),
        pltpu.VMEM((L,), jnp.float32),
    ],
    compiler_params=_SC_PARAMS,
)
def _bin123f_kernel(ps_hbm, d1_hbm, d2_hbm, d3_hbm, s0_hbm, n0_hbm, out_hbm,
                    ps_v, d_v, s_v, n_v, sh_s, sh_n, red_s, red_n,
                    b0s_v, b0n_v, o_v):
    cid = lax.axis_index("c")
    sid = lax.axis_index("s")

    @pl.when(cid == 0)
    def _():
        b = sid // 4
        seg = sid % 4
        zero = jnp.zeros((L,), jnp.float32)
        ones = jnp.ones((L,), jnp.float32)
        iota = lax.iota(jnp.int32, L)
        for j in range(8):
            s_v[pl.ds(j * L, L)] = zero
            n_v[pl.ds(j * L, L)] = zero
        pltpu.sync_copy(ps_hbm.at[b, pl.ds(seg * (32 * HF), 32 * HF)], ps_v)

        pltpu.sync_copy(d1_hbm.at[b, pl.ds(seg * 1024, 1024)], d_v)

        def s1_body(v, carry):
            q = v * L + iota
            pidx = ((q >> 6) << 10) + ((q & 63) << 3)
            p = plsc.load_gather(ps_v, [pidx]) + 32
            d = d_v[pl.ds(v * L, L)]
            plsc.addupdate_scatter(s_v, [p], d)
            plsc.addupdate_scatter(n_v, [p], ones)
            return carry

        lax.fori_loop(0, 1024 // L, s1_body, 0)

        pltpu.sync_copy(d2_hbm.at[b, pl.ds(seg * 256, 256)],
                        d_v.at[pl.ds(0, 256)])

        def s2_body(v, carry):
            q = v * L + iota
            pidx = ((q >> 5) << 11) + ((q & 31) << 4)
            p = plsc.load_gather(ps_v, [pidx]) + 64
            d = d_v[pl.ds(v * L, L)]
            plsc.addupdate_scatter(s_v, [p], d)
            plsc.addupdate_scatter(n_v, [p], ones)
            return carry

        lax.fori_loop(0, 256 // L, s2_body, 0)

        pltpu.sync_copy(d3_hbm.at[b, pl.ds(seg * 64, 64)],
                        d_v.at[pl.ds(0, 64)])

        def s3_body(v, carry):
            q = v * L + iota
            pidx = ((q >> 4) << 12) + ((q & 15) << 5)
            p = plsc.load_gather(ps_v, [pidx]) + 96
            d = d_v[pl.ds(v * L, L)]
            plsc.addupdate_scatter(s_v, [p], d)
            plsc.addupdate_scatter(n_v, [p], ones)
            return carry

        lax.fori_loop(0, 64 // L, s3_body, 0)

        pltpu.sync_copy(s_v, sh_s.at[sid])
        pltpu.sync_copy(n_v, sh_n.at[sid])
        plsc.subcore_barrier()

        @pl.when(sid == 0)
        def _():
            pltpu.sync_copy(sh_s, red_s)
            pltpu.sync_copy(sh_n, red_n)
            pltpu.sync_copy(s0_hbm, b0s_v)
            pltpu.sync_copy(n0_hbm, b0n_v)
            chans = [128.0, 256.0, 512.0, 512.0]
            wts = [1.0, 2.0, 3.0, 4.0]
            total = jnp.zeros((L,), jnp.float32)
            for s in range(4):
                for half in range(2):
                    ssum = jnp.zeros((L,), jnp.float32)
                    nsum = jnp.zeros((L,), jnp.float32)
                    if s == 0:
                        off = half * L
                        for w in range(NW):
                            ssum = ssum + b0s_v[pl.ds(w * 32 + off, L)]
                            nsum = nsum + b0n_v[pl.ds(w * 32 + off, L)]
                    else:
                        off = s * 32 + half * L
                        for w in range(NS):
                            ssum = ssum + red_s[w, pl.ds(off, L)]
                            nsum = nsum + red_n[w, pl.ds(off, L)]
                    cl = half * L + iota
                    coef = jnp.where(
                        cl == 0, jnp.float32(NUM_OLD / NUM_CLASS),
                        jnp.where(cl <= NUM_OLD, jnp.float32(1.0),
                                  jnp.float32(0.0)))
                    denom = jnp.maximum(nsum * chans[s], 1.0)
                    term = jnp.where(nsum > 0, coef * ssum / denom,
                                     jnp.float32(0.0))
                    total = total + wts[s] * term
            o_v[...] = jnp.full((L,), jnp.sum(total) * 0.25)
            pltpu.sync_copy(o_v, out_hbm)


def kernel(labels, outputs_old, f0_old, f1_old, f2_old, f3_old,
           f0, f1, f2, f3, num_class, num_old_class):
    del num_class, num_old_class
    labels = labels.astype(jnp.int32)
    pseudo = _pseudo_tc(outputs_old, labels)

    d1f, d2f, d3f = _dsum_sc(
        f1.transpose(0, 2, 3, 1).reshape(-1),
        f1_old.transpose(0, 2, 3, 1).reshape(-1),
        f2.transpose(0, 2, 3, 1).reshape(-1),
        f2_old.transpose(0, 2, 3, 1).reshape(-1),
        f3.transpose(0, 2, 3, 1).reshape(-1),
        f3_old.transpose(0, 2, 3, 1).reshape(-1),
    )
    d0 = _dsum(f0, f0_old, 16)
    s0, n0 = _bin0_kernel(pseudo, d0)

    out16 = _bin123f_kernel(pseudo, d1f.reshape(B, 4096),
                            d2f.reshape(B, 1024), d3f.reshape(B, 256),
                            s0, n0)
    return out16[0]


# speedup vs baseline: 3.3315x; 3.3315x over previous
import functools

import jax
import jax.numpy as jnp
from jax import lax
from jax.experimental import pallas as pl
from jax.experimental.pallas import tpu as pltpu
from jax.experimental.pallas import tpu_sc as plsc

NC, NS, L = 2, 16, 16
NW = NC * NS

B = 4
HF = 512
OC = 16
H0 = 128
NUM_CLASS = 21
NUM_OLD = 16

_mesh = lambda: plsc.VectorSubcoreMesh(core_axis_name="c", subcore_axis_name="s")
_SC_PARAMS = pltpu.CompilerParams(use_tc_tiling_on_sc=False,
                                  needs_layout_passes=False)


def _dsum_body(f_ref, g_ref, o_ref):
    c = pl.program_id(1)

    @pl.when(c == 0)
    def _():
        o_ref[...] = jnp.zeros_like(o_ref)

    x = f_ref[...] - g_ref[...]
    o_ref[...] += jnp.sum(x * x, axis=1, keepdims=True)


def _dsum(f, f_old, c_blk):
    b, c, h, w = f.shape
    out = pl.pallas_call(
        _dsum_body,
        grid=(b, c // c_blk),
        in_specs=[
            pl.BlockSpec((1, c_blk, h, w), lambda i, j: (i, j, 0, 0)),
            pl.BlockSpec((1, c_blk, h, w), lambda i, j: (i, j, 0, 0)),
        ],
        out_specs=pl.BlockSpec((1, 1, h, w), lambda i, j: (i, 0, 0, 0)),
        out_shape=jax.ShapeDtypeStruct((b, 1, h, w), jnp.float32),
        compiler_params=pltpu.CompilerParams(
            dimension_semantics=("parallel", "arbitrary")),
    )(f, f_old)
    return out.reshape(b, h * w)


def _dsum_cl_body(f_ref, g_ref, o_ref):
    x = f_ref[...] - g_ref[...]
    o_ref[0] = jnp.sum(x * x, axis=2)


def _dsum_cl(f, f_old, hw_blk):
    b, c, h, w = f.shape
    hw = h * w
    f2 = f.transpose(0, 2, 3, 1).reshape(b, hw, c)
    g2 = f_old.transpose(0, 2, 3, 1).reshape(b, hw, c)
    out = pl.pallas_call(
        _dsum_cl_body,
        grid=(b, hw // hw_blk),
        in_specs=[
            pl.BlockSpec((1, hw_blk, c), lambda i, j: (i, j, 0)),
            pl.BlockSpec((1, hw_blk, c), lambda i, j: (i, j, 0)),
        ],
        out_specs=pl.BlockSpec((1, 1, hw_blk), lambda i, j: (i, 0, j)),
        out_shape=jax.ShapeDtypeStruct((b, 1, hw), jnp.float32),
        compiler_params=pltpu.CompilerParams(
            dimension_semantics=("parallel", "parallel")),
    )(f2, g2)
    return out.reshape(b, hw)


_RB = 64


def _pseudo_body(oo_ref, lab_ref, o_ref):
    v = jnp.concatenate(
        [oo_ref[0, :, 4 * k:4 * k + 1, :] for k in range(_RB // 4)], axis=1)
    v = jnp.where(v < 0.5, jnp.float32(0.0), v)
    best = v[0]
    bidx = jnp.zeros((_RB // 4, HF), jnp.int32)
    for c in range(1, OC):
        take = v[c] > best
        best = jnp.where(take, v[c], best)
        bidx = jnp.where(take, c, bidx)
    lab = jnp.concatenate(
        [lab_ref[0, 4 * k:4 * k + 1, :] for k in range(_RB // 4)], axis=0)
    o_ref[0, 0] = jnp.where(lab == 0, bidx, lab)


def _pseudo_tc(outputs_old, labels):
    out = pl.pallas_call(
        _pseudo_body,
        grid=(B, HF // _RB),
        in_specs=[
            pl.BlockSpec((1, OC, _RB, HF), lambda b, r: (b, 0, r, 0)),
            pl.BlockSpec((1, _RB, HF), lambda b, r: (b, r, 0)),
        ],
        out_specs=pl.BlockSpec((1, 1, _RB // 4, HF), lambda b, r: (b, r, 0, 0)),
        out_shape=jax.ShapeDtypeStruct((B, HF // _RB, _RB // 4, HF),
                                       jnp.int32),
        compiler_params=pltpu.CompilerParams(
            dimension_semantics=("parallel", "parallel")),
    )(outputs_old, labels)
    return out.reshape(B, H0 * HF)


@functools.partial(
    pl.kernel,
    out_type=(
        jax.ShapeDtypeStruct((NW * 32,), jnp.float32),
        jax.ShapeDtypeStruct((NW * 32,), jnp.float32),
    ),
    mesh=_mesh(),
    scratch_types=[
        pltpu.VMEM((16 * HF,), jnp.int32),
        pltpu.VMEM((2048,), jnp.float32),
        pltpu.VMEM((32,), jnp.float32),
        pltpu.VMEM((32,), jnp.float32),
    ],
    compiler_params=_SC_PARAMS,
)
def _bin0_kernel(ps_hbm, d0_hbm, s_hbm, n_hbm, ps_v, d_v, s_v, n_v):
    wid = lax.axis_index("s") * NC + lax.axis_index("c")
    b = wid // 8
    seg = wid % 8
    zero = jnp.zeros((L,), jnp.float32)
    ones = jnp.ones((L,), jnp.float32)
    iota = lax.iota(jnp.int32, L)
    s_v[pl.ds(0, L)] = zero
    s_v[pl.ds(L, L)] = zero
    n_v[pl.ds(0, L)] = zero
    n_v[pl.ds(L, L)] = zero
    pltpu.sync_copy(ps_hbm.at[b, pl.ds(seg * (16 * HF), 16 * HF)], ps_v)
    pltpu.sync_copy(d0_hbm.at[b, pl.ds(seg * 2048, 2048)], d_v)

    def s0_body(v, carry):
        q = v * L + iota
        pidx = ((q >> 7) << 9) + ((q & 127) << 2)
        p = plsc.load_gather(ps_v, [pidx])
        d = d_v[pl.ds(v * L, L)]
        plsc.addupdate_scatter(s_v, [p], d)
        plsc.addupdate_scatter(n_v, [p], ones)
        return carry

    lax.fori_loop(0, 2048 // L, s0_body, 0)
    pltpu.sync_copy(s_v, s_hbm.at[pl.ds(wid * 32, 32)])
    pltpu.sync_copy(n_v, n_hbm.at[pl.ds(wid * 32, 32)])


@functools.partial(
    pl.kernel,
    out_type=jax.ShapeDtypeStruct((L,), jnp.float32),
    mesh=_mesh(),
    scratch_types=[
        pltpu.VMEM((32 * HF,), jnp.int32),
        pltpu.VMEM((1024,), jnp.float32),
        pltpu.VMEM((128,), jnp.float32),
        pltpu.VMEM((128,), jnp.float32),
        pltpu.VMEM_SHARED((NS, 128), jnp.float32),
        pltpu.VMEM_SHARED((NS, 128), jnp.float32),
        pltpu.VMEM((NS, 128), jnp.float32),
        pltpu.VMEM((NS, 128), jnp.float32),
        pltpu.VMEM((NW * 32,), jnp.float32),
        pltpu.VMEM((NW * 32,), jnp.float32),
        pltpu.VMEM((L,), jnp.float32),
    ],
    compiler_params=_SC_PARAMS,
)
def _bin123f_kernel(ps_hbm, d1_hbm, d2_hbm, d3_hbm, s0_hbm, n0_hbm, out_hbm,
                    ps_v, d_v, s_v, n_v, sh_s, sh_n, red_s, red_n,
                    b0s_v, b0n_v, o_v):
    cid = lax.axis_index("c")
    sid = lax.axis_index("s")

    @pl.when(cid == 0)
    def _():
        b = sid // 4
        seg = sid % 4
        zero = jnp.zeros((L,), jnp.float32)
        ones = jnp.ones((L,), jnp.float32)
        iota = lax.iota(jnp.int32, L)
        for j in range(8):
            s_v[pl.ds(j * L, L)] = zero
            n_v[pl.ds(j * L, L)] = zero
        pltpu.sync_copy(ps_hbm.at[b, pl.ds(seg * (32 * HF), 32 * HF)], ps_v)

        pltpu.sync_copy(d1_hbm.at[b, pl.ds(seg * 1024, 1024)], d_v)

        def s1_body(v, carry):
            q = v * L + iota
            pidx = ((q >> 6) << 10) + ((q & 63) << 3)
            p = plsc.load_gather(ps_v, [pidx]) + 32
            d = d_v[pl.ds(v * L, L)]
            plsc.addupdate_scatter(s_v, [p], d)
            plsc.addupdate_scatter(n_v, [p], ones)
            return carry

        lax.fori_loop(0, 1024 // L, s1_body, 0)

        pltpu.sync_copy(d2_hbm.at[b, pl.ds(seg * 256, 256)],
                        d_v.at[pl.ds(0, 256)])

        def s2_body(v, carry):
            q = v * L + iota
            pidx = ((q >> 5) << 11) + ((q & 31) << 4)
            p = plsc.load_gather(ps_v, [pidx]) + 64
            d = d_v[pl.ds(v * L, L)]
            plsc.addupdate_scatter(s_v, [p], d)
            plsc.addupdate_scatter(n_v, [p], ones)
            return carry

        lax.fori_loop(0, 256 // L, s2_body, 0)

        pltpu.sync_copy(d3_hbm.at[b, pl.ds(seg * 64, 64)],
                        d_v.at[pl.ds(0, 64)])

        def s3_body(v, carry):
            q = v * L + iota
            pidx = ((q >> 4) << 12) + ((q & 15) << 5)
            p = plsc.load_gather(ps_v, [pidx]) + 96
            d = d_v[pl.ds(v * L, L)]
            plsc.addupdate_scatter(s_v, [p], d)
            plsc.addupdate_scatter(n_v, [p], ones)
            return carry

        lax.fori_loop(0, 64 // L, s3_body, 0)

        pltpu.sync_copy(s_v, sh_s.at[sid])
        pltpu.sync_copy(n_v, sh_n.at[sid])
        plsc.subcore_barrier()

        @pl.when(sid == 0)
        def _():
            pltpu.sync_copy(sh_s, red_s)
            pltpu.sync_copy(sh_n, red_n)
            pltpu.sync_copy(s0_hbm, b0s_v)
            pltpu.sync_copy(n0_hbm, b0n_v)
            chans = [128.0, 256.0, 512.0, 512.0]
            wts = [1.0, 2.0, 3.0, 4.0]
            total = jnp.zeros((L,), jnp.float32)
            for s in range(4):
                for half in range(2):
                    ssum = jnp.zeros((L,), jnp.float32)
                    nsum = jnp.zeros((L,), jnp.float32)
                    if s == 0:
                        off = half * L
                        for w in range(NW):
                            ssum = ssum + b0s_v[pl.ds(w * 32 + off, L)]
                            nsum = nsum + b0n_v[pl.ds(w * 32 + off, L)]
                    else:
                        off = s * 32 + half * L
                        for w in range(NS):
                            ssum = ssum + red_s[w, pl.ds(off, L)]
                            nsum = nsum + red_n[w, pl.ds(off, L)]
                    cl = half * L + iota
                    coef = jnp.where(
                        cl == 0, jnp.float32(NUM_OLD / NUM_CLASS),
                        jnp.where(cl <= NUM_OLD, jnp.float32(1.0),
                                  jnp.float32(0.0)))
                    denom = jnp.maximum(nsum * chans[s], 1.0)
                    term = jnp.where(nsum > 0, coef * ssum / denom,
                                     jnp.float32(0.0))
                    total = total + wts[s] * term
            o_v[...] = jnp.full((L,), jnp.sum(total) * 0.25)
            pltpu.sync_copy(o_v, out_hbm)


def kernel(labels, outputs_old, f0_old, f1_old, f2_old, f3_old,
           f0, f1, f2, f3, num_class, num_old_class):
    del num_class, num_old_class
    labels = labels.astype(jnp.int32)
    pseudo = _pseudo_tc(outputs_old, labels)

    d0 = _dsum(f0, f0_old, 16)
    s0, n0 = _bin0_kernel(pseudo, d0)
    d1 = _dsum_cl(f1, f1_old, 2048)
    d2 = _dsum_cl(f2, f2_old, 1024)
    d3 = _dsum_cl(f3, f3_old, 256)

    out16 = _bin123f_kernel(pseudo, d1, d2, d3, s0, n0)
    return out16[0]


# speedup vs baseline: 3.5924x; 1.0783x over previous
import functools

import jax
import jax.numpy as jnp
from jax import lax
from jax.experimental import pallas as pl
from jax.experimental.pallas import tpu as pltpu
from jax.experimental.pallas import tpu_sc as plsc

NC, NS, L = 2, 16, 16
NW = NC * NS

B = 4
HF = 512
OC = 16
H0 = 128
NUM_CLASS = 21
NUM_OLD = 16

_mesh = lambda: plsc.VectorSubcoreMesh(core_axis_name="c", subcore_axis_name="s")
_SC_PARAMS = pltpu.CompilerParams(use_tc_tiling_on_sc=False,
                                  needs_layout_passes=False)


def _dsum_body(f_ref, g_ref, o_ref):
    c = pl.program_id(1)

    @pl.when(c == 0)
    def _():
        o_ref[...] = jnp.zeros_like(o_ref)

    x = f_ref[...] - g_ref[...]
    o_ref[...] += jnp.sum(x * x, axis=1, keepdims=True)


def _dsum(f, f_old, c_blk):
    b, c, h, w = f.shape
    out = pl.pallas_call(
        _dsum_body,
        grid=(b, c // c_blk),
        in_specs=[
            pl.BlockSpec((1, c_blk, h, w), lambda i, j: (i, j, 0, 0)),
            pl.BlockSpec((1, c_blk, h, w), lambda i, j: (i, j, 0, 0)),
        ],
        out_specs=pl.BlockSpec((1, 1, h, w), lambda i, j: (i, 0, 0, 0)),
        out_shape=jax.ShapeDtypeStruct((b, 1, h, w), jnp.float32),
        compiler_params=pltpu.CompilerParams(
            dimension_semantics=("parallel", "arbitrary")),
    )(f, f_old)
    return out.reshape(b, h * w)


def _dsum_cl_body(f_ref, g_ref, o_ref):
    x = f_ref[...] - g_ref[...]
    o_ref[0] = jnp.sum(x * x, axis=2)


def _dsum_cl(f, f_old, hw_blk):
    b, c, h, w = f.shape
    hw = h * w
    f2 = f.transpose(0, 2, 3, 1).reshape(b, hw, c)
    g2 = f_old.transpose(0, 2, 3, 1).reshape(b, hw, c)
    out = pl.pallas_call(
        _dsum_cl_body,
        grid=(b, hw // hw_blk),
        in_specs=[
            pl.BlockSpec((1, hw_blk, c), lambda i, j: (i, j, 0)),
            pl.BlockSpec((1, hw_blk, c), lambda i, j: (i, j, 0)),
        ],
        out_specs=pl.BlockSpec((1, 1, hw_blk), lambda i, j: (i, 0, j)),
        out_shape=jax.ShapeDtypeStruct((b, 1, hw), jnp.float32),
        compiler_params=pltpu.CompilerParams(
            dimension_semantics=("parallel", "parallel")),
    )(f2, g2)
    return out.reshape(b, hw)


_RB = 64


def _pseudo_body(oo_ref, lab_ref, o_ref):
    v = jnp.concatenate(
        [oo_ref[0, :, 4 * k:4 * k + 1, :] for k in range(_RB // 4)], axis=1)
    v = jnp.where(v < 0.5, jnp.float32(0.0), v)
    best = v[0]
    bidx = jnp.zeros((_RB // 4, HF), jnp.int32)
    for c in range(1, OC):
        take = v[c] > best
        best = jnp.where(take, v[c], best)
        bidx = jnp.where(take, c, bidx)
    lab = jnp.concatenate(
        [lab_ref[0, 4 * k:4 * k + 1, :] for k in range(_RB // 4)], axis=0)
    o_ref[0, 0] = jnp.where(lab == 0, bidx, lab)


def _pseudo_tc(outputs_old, labels):
    out = pl.pallas_call(
        _pseudo_body,
        grid=(B, HF // _RB),
        in_specs=[
            pl.BlockSpec((1, OC, _RB, HF), lambda b, r: (b, 0, r, 0)),
            pl.BlockSpec((1, _RB, HF), lambda b, r: (b, r, 0)),
        ],
        out_specs=pl.BlockSpec((1, 1, _RB // 4, HF), lambda b, r: (b, r, 0, 0)),
        out_shape=jax.ShapeDtypeStruct((B, HF // _RB, _RB // 4, HF),
                                       jnp.int32),
        compiler_params=pltpu.CompilerParams(
            dimension_semantics=("parallel", "parallel")),
    )(outputs_old, labels)
    return out.reshape(B, H0 * HF)


@functools.partial(
    pl.kernel,
    out_type=(
        jax.ShapeDtypeStruct((NW * 32,), jnp.float32),
        jax.ShapeDtypeStruct((NW * 32,), jnp.float32),
    ),
    mesh=_mesh(),
    scratch_types=[
        pltpu.VMEM((16 * HF,), jnp.int32),
        pltpu.VMEM((2048,), jnp.float32),
        pltpu.VMEM((32,), jnp.float32),
        pltpu.VMEM((32,), jnp.float32),
    ],
    compiler_params=_SC_PARAMS,
)
def _bin0_kernel(ps_hbm, d0_hbm, s_hbm, n_hbm, ps_v, d_v, s_v, n_v):
    wid = lax.axis_index("s") * NC + lax.axis_index("c")
    b = wid // 8
    seg = wid % 8
    zero = jnp.zeros((L,), jnp.float32)
    ones = jnp.ones((L,), jnp.float32)
    iota = lax.iota(jnp.int32, L)
    s_v[pl.ds(0, L)] = zero
    s_v[pl.ds(L, L)] = zero
    n_v[pl.ds(0, L)] = zero
    n_v[pl.ds(L, L)] = zero
    pltpu.sync_copy(ps_hbm.at[b, pl.ds(seg * (16 * HF), 16 * HF)], ps_v)
    pltpu.sync_copy(d0_hbm.at[b, pl.ds(seg * 2048, 2048)], d_v)

    def s0_body(v, carry):
        q = v * L + iota
        pidx = ((q >> 7) << 9) + ((q & 127) << 2)
        p = plsc.load_gather(ps_v, [pidx])
        d = d_v[pl.ds(v * L, L)]
        plsc.addupdate_scatter(s_v, [p], d)
        plsc.addupdate_scatter(n_v, [p], ones)
        return carry

    lax.fori_loop(0, 2048 // L, s0_body, 0)
    pltpu.sync_copy(s_v, s_hbm.at[pl.ds(wid * 32, 32)])
    pltpu.sync_copy(n_v, n_hbm.at[pl.ds(wid * 32, 32)])


@functools.partial(
    pl.kernel,
    out_type=jax.ShapeDtypeStruct((L,), jnp.float32),
    mesh=_mesh(),
    scratch_types=[
        pltpu.VMEM((32 * HF,), jnp.int32),
        pltpu.VMEM((1024,), jnp.float32),
        pltpu.VMEM((128,), jnp.float32),
        pltpu.VMEM((128,), jnp.float32),
        pltpu.VMEM_SHARED((NS, 128), jnp.float32),
        pltpu.VMEM_SHARED((NS, 128), jnp.float32),
        pltpu.VMEM((NS, 128), jnp.float32),
        pltpu.VMEM((NS, 128), jnp.float32),
        pltpu.VMEM((NW * 32,), jnp.float32),
        pltpu.VMEM((NW * 32,), jnp.float32),
        pltpu.VMEM((L,), jnp.float32),
    ],
    compiler_params=_SC_PARAMS,
)
def _bin123f_kernel(ps_hbm, d1_hbm, d2_hbm, d3_hbm, s0_hbm, n0_hbm, out_hbm,
                    ps_v, d_v, s_v, n_v, sh_s, sh_n, red_s, red_n,
                    b0s_v, b0n_v, o_v):
    cid = lax.axis_index("c")
    sid = lax.axis_index("s")

    @pl.when(cid == 0)
    def _():
        b = sid // 4
        seg = sid % 4
        zero = jnp.zeros((L,), jnp.float32)
        ones = jnp.ones((L,), jnp.float32)
        iota = lax.iota(jnp.int32, L)
        for j in range(8):
            s_v[pl.ds(j * L, L)] = zero
            n_v[pl.ds(j * L, L)] = zero
        pltpu.sync_copy(ps_hbm.at[b, pl.ds(seg * (32 * HF), 32 * HF)], ps_v)

        pltpu.sync_copy(d1_hbm.at[b, pl.ds(seg * 1024, 1024)], d_v)

        def s1_body(v, carry):
            q = v * L + iota
            pidx = ((q >> 6) << 10) + ((q & 63) << 3)
            p = plsc.load_gather(ps_v, [pidx]) + 32
            d = d_v[pl.ds(v * L, L)]
            plsc.addupdate_scatter(s_v, [p], d)
            plsc.addupdate_scatter(n_v, [p], ones)
            return carry

        lax.fori_loop(0, 1024 // L, s1_body, 0)

        pltpu.sync_copy(d2_hbm.at[b, pl.ds(seg * 256, 256)],
                        d_v.at[pl.ds(0, 256)])

        def s2_body(v, carry):
            q = v * L + iota
            pidx = ((q >> 5) << 11) + ((q & 31) << 4)
            p = plsc.load_gather(ps_v, [pidx]) + 64
            d = d_v[pl.ds(v * L, L)]
            plsc.addupdate_scatter(s_v, [p], d)
            plsc.addupdate_scatter(n_v, [p], ones)
            return carry

        lax.fori_loop(0, 256 // L, s2_body, 0)

        pltpu.sync_copy(d3_hbm.at[b, pl.ds(seg * 64, 64)],
                        d_v.at[pl.ds(0, 64)])

        def s3_body(v, carry):
            q = v * L + iota
            pidx = ((q >> 4) << 12) + ((q & 15) << 5)
            p = plsc.load_gather(ps_v, [pidx]) + 96
            d = d_v[pl.ds(v * L, L)]
            plsc.addupdate_scatter(s_v, [p], d)
            plsc.addupdate_scatter(n_v, [p], ones)
            return carry

        lax.fori_loop(0, 64 // L, s3_body, 0)

        pltpu.sync_copy(s_v, sh_s.at[sid])
        pltpu.sync_copy(n_v, sh_n.at[sid])
        plsc.subcore_barrier()

        @pl.when(sid == 0)
        def _():
            pltpu.sync_copy(sh_s, red_s)
            pltpu.sync_copy(sh_n, red_n)
            pltpu.sync_copy(s0_hbm, b0s_v)
            pltpu.sync_copy(n0_hbm, b0n_v)
            chans = [128.0, 256.0, 512.0, 512.0]
            wts = [1.0, 2.0, 3.0, 4.0]
            total = jnp.zeros((L,), jnp.float32)
            for s in range(4):
                for half in range(2):
                    ssum = jnp.zeros((L,), jnp.float32)
                    nsum = jnp.zeros((L,), jnp.float32)
                    if s == 0:
                        off = half * L
                        for w in range(NW):
                            ssum = ssum + b0s_v[pl.ds(w * 32 + off, L)]
                            nsum = nsum + b0n_v[pl.ds(w * 32 + off, L)]
                    else:
                        off = s * 32 + half * L
                        for w in range(NS):
                            ssum = ssum + red_s[w, pl.ds(off, L)]
                            nsum = nsum + red_n[w, pl.ds(off, L)]
                    cl = half * L + iota
                    coef = jnp.where(
                        cl == 0, jnp.float32(NUM_OLD / NUM_CLASS),
                        jnp.where(cl <= NUM_OLD, jnp.float32(1.0),
                                  jnp.float32(0.0)))
                    denom = jnp.maximum(nsum * chans[s], 1.0)
                    term = jnp.where(nsum > 0, coef * ssum / denom,
                                     jnp.float32(0.0))
                    total = total + wts[s] * term
            o_v[...] = jnp.full((L,), jnp.sum(total) * 0.25)
            pltpu.sync_copy(o_v, out_hbm)


def kernel(labels, outputs_old, f0_old, f1_old, f2_old, f3_old,
           f0, f1, f2, f3, num_class, num_old_class):
    del num_class, num_old_class
    labels = labels.astype(jnp.int32)
    pseudo = _pseudo_tc(outputs_old, labels)

    d0 = _dsum(f0, f0_old, 32)
    s0, n0 = _bin0_kernel(pseudo, d0)
    d1 = _dsum_cl(f1, f1_old, 4096)
    d2 = _dsum_cl(f2, f2_old, 1024)
    d3 = _dsum_cl(f3, f3_old, 256)

    out16 = _bin123f_kernel(pseudo, d1, d2, d3, s0, n0)
    return out16[0]


# speedup vs baseline: 3.6138x; 1.0060x over previous
import functools

import jax
import jax.numpy as jnp
from jax import lax
from jax.experimental import pallas as pl
from jax.experimental.pallas import tpu as pltpu
from jax.experimental.pallas import tpu_sc as plsc

NC, NS, L = 2, 16, 16
NW = NC * NS

B = 4
HF = 512
OC = 16
H0 = 128
NUM_CLASS = 21
NUM_OLD = 16

_mesh = lambda: plsc.VectorSubcoreMesh(core_axis_name="c", subcore_axis_name="s")
_SC_PARAMS = pltpu.CompilerParams(use_tc_tiling_on_sc=False,
                                  needs_layout_passes=False)


def _dsum_body(f_ref, g_ref, o_ref):
    c = pl.program_id(1)

    @pl.when(c == 0)
    def _():
        o_ref[...] = jnp.zeros_like(o_ref)

    x = f_ref[...] - g_ref[...]
    o_ref[...] += jnp.sum(x * x, axis=1, keepdims=True)


def _dsum(f, f_old, c_blk):
    b, c, h, w = f.shape
    out = pl.pallas_call(
        _dsum_body,
        grid=(b, c // c_blk),
        in_specs=[
            pl.BlockSpec((1, c_blk, h, w), lambda i, j: (i, j, 0, 0)),
            pl.BlockSpec((1, c_blk, h, w), lambda i, j: (i, j, 0, 0)),
        ],
        out_specs=pl.BlockSpec((1, 1, h, w), lambda i, j: (i, 0, 0, 0)),
        out_shape=jax.ShapeDtypeStruct((b, 1, h, w), jnp.float32),
        compiler_params=pltpu.CompilerParams(
            dimension_semantics=("parallel", "arbitrary")),
    )(f, f_old)
    return out.reshape(b, h * w)


def _dsum_cl_body(f_ref, g_ref, o_ref):
    x = f_ref[...] - g_ref[...]
    o_ref[0] = jnp.sum(x * x, axis=2)


def _dsum_cl(f, f_old, hw_blk):
    b, c, h, w = f.shape
    hw = h * w
    f2 = f.transpose(0, 2, 3, 1).reshape(b, hw, c)
    g2 = f_old.transpose(0, 2, 3, 1).reshape(b, hw, c)
    out = pl.pallas_call(
        _dsum_cl_body,
        grid=(b, hw // hw_blk),
        in_specs=[
            pl.BlockSpec((1, hw_blk, c), lambda i, j: (i, j, 0)),
            pl.BlockSpec((1, hw_blk, c), lambda i, j: (i, j, 0)),
        ],
        out_specs=pl.BlockSpec((1, 1, hw_blk), lambda i, j: (i, 0, j)),
        out_shape=jax.ShapeDtypeStruct((b, 1, hw), jnp.float32),
        compiler_params=pltpu.CompilerParams(
            dimension_semantics=("parallel", "parallel")),
    )(f2, g2)
    return out.reshape(b, hw)


_RB = 64


def _pseudo_body(oo_ref, lab_ref, o_ref):
    v = jnp.concatenate(
        [oo_ref[0, :, 4 * k:4 * k + 1, :] for k in range(_RB // 4)], axis=1)
    v = jnp.where(v < 0.5, jnp.float32(0.0), v)
    best = v[0]
    bidx = jnp.zeros((_RB // 4, HF), jnp.int32)
    for c in range(1, OC):
        take = v[c] > best
        best = jnp.where(take, v[c], best)
        bidx = jnp.where(take, c, bidx)
    lab = jnp.concatenate(
        [lab_ref[0, 4 * k:4 * k + 1, :] for k in range(_RB // 4)], axis=0)
    o_ref[0, 0] = jnp.where(lab == 0, bidx, lab)


def _pseudo_tc(outputs_old, labels):
    out = pl.pallas_call(
        _pseudo_body,
        grid=(B, HF // _RB),
        in_specs=[
            pl.BlockSpec((1, OC, _RB, HF), lambda b, r: (b, 0, r, 0)),
            pl.BlockSpec((1, _RB, HF), lambda b, r: (b, r, 0)),
        ],
        out_specs=pl.BlockSpec((1, 1, _RB // 4, HF), lambda b, r: (b, r, 0, 0)),
        out_shape=jax.ShapeDtypeStruct((B, HF // _RB, _RB // 4, HF),
                                       jnp.int32),
        compiler_params=pltpu.CompilerParams(
            dimension_semantics=("parallel", "parallel")),
    )(outputs_old, labels)
    return out.reshape(B, H0 * HF)


@functools.partial(
    pl.kernel,
    out_type=(
        jax.ShapeDtypeStruct((NW * 32,), jnp.float32),
        jax.ShapeDtypeStruct((NW * 32,), jnp.float32),
    ),
    mesh=_mesh(),
    scratch_types=[
        pltpu.VMEM((16 * HF,), jnp.int32),
        pltpu.VMEM((2048,), jnp.float32),
        pltpu.VMEM((32,), jnp.float32),
        pltpu.VMEM((32,), jnp.float32),
    ],
    compiler_params=_SC_PARAMS,
)
def _bin0_kernel(ps_hbm, d0_hbm, s_hbm, n_hbm, ps_v, d_v, s_v, n_v):
    wid = lax.axis_index("s") * NC + lax.axis_index("c")
    b = wid // 8
    seg = wid % 8
    zero = jnp.zeros((L,), jnp.float32)
    ones = jnp.ones((L,), jnp.float32)
    iota = lax.iota(jnp.int32, L)
    s_v[pl.ds(0, L)] = zero
    s_v[pl.ds(L, L)] = zero
    n_v[pl.ds(0, L)] = zero
    n_v[pl.ds(L, L)] = zero
    pltpu.sync_copy(ps_hbm.at[b, pl.ds(seg * (16 * HF), 16 * HF)], ps_v)
    pltpu.sync_copy(d0_hbm.at[b, pl.ds(seg * 2048, 2048)], d_v)

    def s0_body(v, carry):
        q = v * L + iota
        pidx = ((q >> 7) << 9) + ((q & 127) << 2)
        p = plsc.load_gather(ps_v, [pidx])
        d = d_v[pl.ds(v * L, L)]
        plsc.addupdate_scatter(s_v, [p], d)
        plsc.addupdate_scatter(n_v, [p], ones)
        return carry

    lax.fori_loop(0, 2048 // L, s0_body, 0)
    pltpu.sync_copy(s_v, s_hbm.at[pl.ds(wid * 32, 32)])
    pltpu.sync_copy(n_v, n_hbm.at[pl.ds(wid * 32, 32)])


@functools.partial(
    pl.kernel,
    out_type=jax.ShapeDtypeStruct((L,), jnp.float32),
    mesh=_mesh(),
    scratch_types=[
        pltpu.VMEM((32 * HF,), jnp.int32),
        pltpu.VMEM((1024,), jnp.float32),
        pltpu.VMEM((128,), jnp.float32),
        pltpu.VMEM((128,), jnp.float32),
        pltpu.VMEM_SHARED((NS, 128), jnp.float32),
        pltpu.VMEM_SHARED((NS, 128), jnp.float32),
        pltpu.VMEM((NS, 128), jnp.float32),
        pltpu.VMEM((NS, 128), jnp.float32),
        pltpu.VMEM((NW * 32,), jnp.float32),
        pltpu.VMEM((NW * 32,), jnp.float32),
        pltpu.VMEM((L,), jnp.float32),
    ],
    compiler_params=_SC_PARAMS,
)
def _bin123f_kernel(ps_hbm, d1_hbm, d2_hbm, d3_hbm, s0_hbm, n0_hbm, out_hbm,
                    ps_v, d_v, s_v, n_v, sh_s, sh_n, red_s, red_n,
                    b0s_v, b0n_v, o_v):
    cid = lax.axis_index("c")
    sid = lax.axis_index("s")

    @pl.when(cid == 0)
    def _():
        b = sid // 4
        seg = sid % 4
        zero = jnp.zeros((L,), jnp.float32)
        ones = jnp.ones((L,), jnp.float32)
        iota = lax.iota(jnp.int32, L)
        for j in range(8):
            s_v[pl.ds(j * L, L)] = zero
            n_v[pl.ds(j * L, L)] = zero
        pltpu.sync_copy(ps_hbm.at[b, pl.ds(seg * (32 * HF), 32 * HF)], ps_v)

        pltpu.sync_copy(d1_hbm.at[b, pl.ds(seg * 1024, 1024)], d_v)

        def s1_body(v, carry):
            q = v * L + iota
            pidx = ((q >> 6) << 10) + ((q & 63) << 3)
            p = plsc.load_gather(ps_v, [pidx]) + 32
            d = d_v[pl.ds(v * L, L)]
            plsc.addupdate_scatter(s_v, [p], d)
            plsc.addupdate_scatter(n_v, [p], ones)
            return carry

        lax.fori_loop(0, 1024 // L, s1_body, 0)

        pltpu.sync_copy(d2_hbm.at[b, pl.ds(seg * 256, 256)],
                        d_v.at[pl.ds(0, 256)])

        def s2_body(v, carry):
            q = v * L + iota
            pidx = ((q >> 5) << 11) + ((q & 31) << 4)
            p = plsc.load_gather(ps_v, [pidx]) + 64
            d = d_v[pl.ds(v * L, L)]
            plsc.addupdate_scatter(s_v, [p], d)
            plsc.addupdate_scatter(n_v, [p], ones)
            return carry

        lax.fori_loop(0, 256 // L, s2_body, 0)

        pltpu.sync_copy(d3_hbm.at[b, pl.ds(seg * 64, 64)],
                        d_v.at[pl.ds(0, 64)])

        def s3_body(v, carry):
            q = v * L + iota
            pidx = ((q >> 4) << 12) + ((q & 15) << 5)
            p = plsc.load_gather(ps_v, [pidx]) + 96
            d = d_v[pl.ds(v * L, L)]
            plsc.addupdate_scatter(s_v, [p], d)
            plsc.addupdate_scatter(n_v, [p], ones)
            return carry

        lax.fori_loop(0, 64 // L, s3_body, 0)

        pltpu.sync_copy(s_v, sh_s.at[sid])
        pltpu.sync_copy(n_v, sh_n.at[sid])
        plsc.subcore_barrier()

        @pl.when(sid == 0)
        def _():
            pltpu.sync_copy(sh_s, red_s)
            pltpu.sync_copy(sh_n, red_n)
            pltpu.sync_copy(s0_hbm, b0s_v)
            pltpu.sync_copy(n0_hbm, b0n_v)
            chans = [128.0, 256.0, 512.0, 512.0]
            wts = [1.0, 2.0, 3.0, 4.0]
            total = jnp.zeros((L,), jnp.float32)
            for s in range(4):
                for half in range(2):
                    ssum = jnp.zeros((L,), jnp.float32)
                    nsum = jnp.zeros((L,), jnp.float32)
                    if s == 0:
                        off = half * L
                        for w in range(NW):
                            ssum = ssum + b0s_v[pl.ds(w * 32 + off, L)]
                            nsum = nsum + b0n_v[pl.ds(w * 32 + off, L)]
                    else:
                        off = s * 32 + half * L
                        for w in range(NS):
                            ssum = ssum + red_s[w, pl.ds(off, L)]
                            nsum = nsum + red_n[w, pl.ds(off, L)]
                    cl = half * L + iota
                    coef = jnp.where(
                        cl == 0, jnp.float32(NUM_OLD / NUM_CLASS),
                        jnp.where(cl <= NUM_OLD, jnp.float32(1.0),
                                  jnp.float32(0.0)))
                    denom = jnp.maximum(nsum * chans[s], 1.0)
                    term = jnp.where(nsum > 0, coef * ssum / denom,
                                     jnp.float32(0.0))
                    total = total + wts[s] * term
            o_v[...] = jnp.full((L,), jnp.sum(total) * 0.25)
            pltpu.sync_copy(o_v, out_hbm)


def kernel(labels, outputs_old, f0_old, f1_old, f2_old, f3_old,
           f0, f1, f2, f3, num_class, num_old_class):
    del num_class, num_old_class
    labels = labels.astype(jnp.int32)
    pseudo = _pseudo_tc(outputs_old, labels)

    d0 = _dsum(f0, f0_old, 64)
    s0, n0 = _bin0_kernel(pseudo, d0)
    d1 = _dsum_cl(f1, f1_old, 4096)
    d2 = _dsum_cl(f2, f2_old, 1024)
    d3 = _dsum_cl(f3, f3_old, 256)

    out16 = _bin123f_kernel(pseudo, d1, d2, d3, s0, n0)
    return out16[0]
